# baseline (speedup 1.0000x reference)
import jax
import jax.numpy as jnp
from jax import lax
from jax.experimental import pallas as pl
from jax.experimental.pallas import tpu as pltpu
from jax.experimental.pallas import tpu_sc as plsc

_W = 128
_LB = 8
_NW = 32


def _gather_t(weight_pairs, idx_t, n_l, n_b, d):
    n_pairs = weight_pairs.shape[0]
    lb_tiles = idx_t.shape[0] // _LB
    n_bb = n_b // _W
    supers = lb_tiles * n_bb
    per_tile = supers // _NW
    mesh = plsc.VectorSubcoreMesh(core_axis_name="core", subcore_axis_name="subcore")

    @pl.kernel(
        out_type=jax.ShapeDtypeStruct((n_l, d, n_b), jnp.float32),
        mesh=mesh,
        scratch_types=[
            pltpu.VMEM((_LB, _W), jnp.int32),
            pltpu.VMEM((_LB, _W), jnp.int32),
            pltpu.VMEM((_LB, _W), jnp.int32),
            pltpu.VMEM((_W, 2 * d), jnp.float32),
            pltpu.VMEM((d, _W), jnp.float32),
            pltpu.SemaphoreType.DMA,
        ],
        compiler_params=pltpu.CompilerParams(
            use_tc_tiling_on_sc=True, needs_layout_passes=False
        ),
    )
    def kern(x_hbm, i_hbm, o_hbm, ir_v, ip_v, of_v, g_v, o_v, sem):
        wid = lax.axis_index("subcore") * 2 + lax.axis_index("core")

        @pl.loop(0, per_tile)
        def _(s):
            sw = wid * per_tile + s
            lb = sw // n_bb
            bb = sw % n_bb
            pltpu.sync_copy(
                i_hbm.at[pl.ds(lb * _LB, _LB), pl.ds(bb * _W, _W)], ir_v
            )
            for lr in range(_LB):
                for t in range(_W // 16):
                    v = ir_v[lr, pl.ds(t * 16, 16)]
                    ip_v[lr, pl.ds(t * 16, 16)] = jnp.minimum(
                        v >> 1, n_pairs - 1
                    )
                    of_v[lr, pl.ds(t * 16, 16)] = (v & 1) * d
            for lr in range(_LB):
                l = lb * _LB + lr

                @pl.when(l < n_l)
                def _():
                    pltpu.async_copy(x_hbm.at[ip_v.at[lr]], g_v, sem).wait()

                    @pl.loop(0, _W // 16)
                    def _(t):
                        rows = t * 16 + lax.iota(jnp.int32, 16)
                        offs = of_v[lr, pl.ds(t * 16, 16)]
                        for f in range(d):
                            o_v[f, pl.ds(t * 16, 16)] = plsc.load_gather(
                                g_v, [rows, offs + f]
                            )

                    pltpu.sync_copy(
                        o_v, o_hbm.at[l, :, pl.ds(bb * _W, _W)]
                    )

    return kern(weight_pairs, idx_t)


def kernel(indices, weight):
    b, l = indices.shape
    d = weight.shape[1]
    idx_t = indices.transpose(1, 0).astype(jnp.int32)
    lb_pad = (l + _LB - 1) // _LB * _LB
    idx_t = jnp.pad(idx_t, ((0, lb_pad - l), (0, 0)))
    weight_pairs = weight.reshape(-1, 2 * d)
    out_t = _gather_t(weight_pairs, idx_t, l, b, d)
    return out_t.transpose(2, 0, 1)


# speedup vs baseline: 1.3662x; 1.3662x over previous
import jax
import jax.numpy as jnp
from jax import lax
from jax.experimental import pallas as pl
from jax.experimental.pallas import tpu as pltpu
from jax.experimental.pallas import tpu_sc as plsc

_W = 128
_J = 2
_NW = 32


def _gather_compact(weight_pairs, idx_flat):
    n = idx_flat.shape[0]
    steps = n // (_NW * _J * _W)
    per_tile_out = n // (2 * _NW)
    mesh = plsc.VectorSubcoreMesh(core_axis_name="core", subcore_axis_name="subcore")

    @pl.kernel(
        out_type=jax.ShapeDtypeStruct((n // 2, 128), jnp.float32),
        mesh=mesh,
        scratch_types=[
            pltpu.VMEM((_J, _W, 128), jnp.float32),
            pltpu.VMEM((_J * _W // 2, 128), jnp.float32),
            pltpu.VMEM((_J, _W), jnp.int32),
            pltpu.VMEM((_J * _W,), jnp.int32),
            pltpu.SemaphoreType.DMA,
            pltpu.SemaphoreType.DMA,
        ],
        compiler_params=pltpu.CompilerParams(use_tc_tiling_on_sc=True),
    )
    def kern(x_hbm, i_hbm, o_hbm, g_v, o_v, ip_v, ir_v, gsem, osem):
        wid = lax.axis_index("subcore") * 2 + lax.axis_index("core")
        idx_base = wid * (steps * _J * _W)
        out_base = wid * per_tile_out

        @pl.loop(0, steps)
        def _(s):
            base = idx_base + s * (_J * _W)
            pltpu.sync_copy(i_hbm.at[pl.ds(base, _J * _W)], ir_v)
            for j in range(_J):
                for t in range(_W // 16):
                    v = ir_v[pl.ds(j * _W + t * 16, 16)]
                    ip_v[j, pl.ds(t * 16, 16)] = v >> 1
            copies = [
                pltpu.async_copy(x_hbm.at[ip_v.at[j]], g_v.at[j], gsem)
                for j in range(_J)
            ]
            for j in range(_J):
                copies[j].wait()

                @pl.loop(0, _W // 16)
                def _(t):
                    hv = ir_v[pl.ds(j * _W + t * 16, 16)] & 1
                    for lane in range(16):
                        off = hv[lane] * 64
                        r = t * 16 + lane
                        rr = j * _W + r
                        k = rr >> 1
                        h = (rr & 1) * 64
                        for c in range(4):
                            o_v[k, pl.ds(h + c * 16, 16)] = (
                                g_v[j, r, pl.ds(off + c * 16, 16)]
                            )

            pltpu.async_copy(
                o_v, o_hbm.at[pl.ds(out_base + s * (_J * _W // 2), _J * _W // 2)],
                osem,
            ).wait()

    return kern(weight_pairs, idx_flat)


def kernel(indices, weight):
    b, l = indices.shape
    d = weight.shape[1]
    idx_flat = indices.reshape(-1).astype(jnp.int32)
    weight_pairs = weight.reshape(-1, 2 * d)
    out_pairs = _gather_compact(weight_pairs, idx_flat)
    return out_pairs.reshape(b, l, d)
